# Initial kernel scaffold; baseline (speedup 1.0000x reference)
import jax
import jax.numpy as jnp
from jax.experimental import pallas as pl


def kernel(feat, edge_index, b, W_rel1, b_rel1, W_root1, W_rel2, b_rel2, W_root2, W1, bb1, W2, bb2, W3, bb3):
    raise NotImplementedError("write your pallas kernel here")


# speedup vs baseline: 6.1828x; 6.1828x over previous
import functools

import jax
import jax.numpy as jnp
from jax import lax
from jax.experimental import pallas as pl
from jax.experimental.pallas import tpu as pltpu
from jax.experimental.pallas import tpu_sc as plsc

NC = 2
NS = 16
NW = NC * NS
CH = 128
DUM = 2048


def _seg_sum_sc(table, src, dst, n, d):
    e = src.shape[0]
    h = n // NC
    epw = e // NS
    nfull = epw // CH
    tail = epw - nfull * CH
    zr = -(-(h + DUM) // (NS * 8)) * 8
    hp = zr * NS
    big = -(-h // NS) // 8 * 8
    while big * (NS - 1) > h:
        big -= 8
    rem = h - big * (NS - 1)
    assert tail % 16 == 0 and rem > 0 and (d * big) % 8 == 0 and (d * rem) % 8 == 0

    zeros = jnp.zeros((zr, d), jnp.float32)

    mesh = plsc.VectorSubcoreMesh(core_axis_name="c", subcore_axis_name="s",
                                  num_cores=NC, num_subcores=NS)

    def body(z_hbm, table_hbm, src_hbm, dst_hbm, out_hbm,
             idx_v, dst_v, rows_v, idx_t, dst_t, rows_t, acc, sem):
        c = lax.axis_index("c")
        s = lax.axis_index("s")
        base_node = c * h

        pltpu.sync_copy(z_hbm, acc.at[pl.ds(s * zr, zr)])
        plsc.subcore_barrier()

        ebase = s * epw

        def remap(dstr, nb):
            for k in range(nb // 16):
                dd = dstr[pl.ds(k * 16, 16)]
                inr = (dd >= base_node) & (dd < base_node + h)
                dl = jnp.where(inr, dd - base_node, h + (dd & (DUM - 1)))
                dstr[pl.ds(k * 16, 16)] = dl

        def chunk(off, nb, idxr, dstr, rowsr):
            off = pl.multiple_of(off, 8)
            pltpu.sync_copy(src_hbm.at[pl.ds(off, nb)], idxr)
            pltpu.sync_copy(dst_hbm.at[pl.ds(off, nb)], dstr)
            remap(dstr, nb)
            pltpu.async_copy(table_hbm.at[idxr], rowsr, sem).wait()
            pltpu.sync_copy(rowsr, acc.at[dstr], add=True)

        def loop_body(i, carry):
            chunk(ebase + i * CH, CH, idx_v, dst_v, rows_v)
            return carry

        lax.fori_loop(0, nfull, loop_body, 0)
        if tail:
            chunk(ebase + nfull * CH, tail, idx_t, dst_t, rows_t)

        plsc.subcore_barrier()
        plsc.subcore_barrier()
        plsc.subcore_barrier()

        @pl.when(s < NS - 1)
        def _():
            o = pl.multiple_of(s * big, 8)
            pltpu.sync_copy(acc.at[pl.ds(o, big)],
                            out_hbm.at[pl.ds(base_node + o, big)])

        @pl.when(s == NS - 1)
        def _():
            o = (NS - 1) * big
            pltpu.sync_copy(acc.at[pl.ds(o, rem)],
                            out_hbm.at[pl.ds(base_node + o, rem)])

    kern = pl.kernel(
        body,
        out_type=jax.ShapeDtypeStruct((n, d), jnp.float32),
        mesh=mesh,
        compiler_params=pltpu.CompilerParams(use_tc_tiling_on_sc=False,
                                             has_side_effects=True),
        scratch_types=[
            pltpu.VMEM((CH,), jnp.int32),
            pltpu.VMEM((CH,), jnp.int32),
            pltpu.VMEM((CH, d), jnp.float32),
            pltpu.VMEM((max(tail, 16),), jnp.int32),
            pltpu.VMEM((max(tail, 16),), jnp.int32),
            pltpu.VMEM((max(tail, 16), d), jnp.float32),
            pltpu.VMEM_SHARED((hp, d), jnp.float32),
            pltpu.SemaphoreType.DMA,
        ],
    )
    return kern(zeros, table, src, dst)


def _dense1(feat, agg1, W_rel1, b_rel1, W_root1, W_rel2, b_rel2, W_root2, blk):
    n = feat.shape[0]
    grid = n // blk

    def body(feat_ref, agg_ref, wr1, br1, wo1, wr2, br2, wo2, g_ref, hr_ref):
        x = feat_ref[...]
        a = agg_ref[...]
        hcur = jnp.maximum(a * wr1[...] + x * wo1[...] + br1[...], 0.0)
        hp = jax.lax.Precision.HIGHEST
        g_ref[...] = jnp.dot(hcur, wr2[...], preferred_element_type=jnp.float32,
                             precision=hp)
        hr_ref[...] = jnp.dot(hcur, wo2[...], preferred_element_type=jnp.float32,
                              precision=hp) + br2[...]

    full = lambda shape: pl.BlockSpec(shape, lambda i: (0, 0))
    return pl.pallas_call(
        body,
        grid=(grid,),
        in_specs=[
            pl.BlockSpec((blk, 1), lambda i: (i, 0)),
            pl.BlockSpec((blk, 1), lambda i: (i, 0)),
            full((1, 64)), full((1, 64)), full((1, 64)),
            full((64, 32)), full((1, 32)), full((64, 32)),
        ],
        out_specs=[pl.BlockSpec((blk, 32), lambda i: (i, 0)),
                   pl.BlockSpec((blk, 32), lambda i: (i, 0))],
        out_shape=[jax.ShapeDtypeStruct((n, 32), jnp.float32),
                   jax.ShapeDtypeStruct((n, 32), jnp.float32)],
    )(feat, agg1, W_rel1.reshape(1, 64), b_rel1.reshape(1, 64), W_root1.reshape(1, 64),
      W_rel2, b_rel2.reshape(1, 32), W_root2)


def _dense2(agg2, hr, b3d, W1, bb1, W2, bb2, W3, bb3, num_graphs, blk):
    n = agg2.shape[0]
    grid = n // blk

    def body(agg_ref, hr_ref, b_ref, w1, c1, w2, c2, w3, c3, out_ref):
        i = pl.program_id(0)
        hp = jax.lax.Precision.HIGHEST
        x2 = jnp.maximum(agg_ref[...] + hr_ref[...], 0.0)
        t = jnp.maximum(jnp.dot(x2, w1[...], preferred_element_type=jnp.float32,
                                precision=hp) + c1[...], 0.0)
        t = jnp.maximum(jnp.dot(t, w2[...], preferred_element_type=jnp.float32,
                                precision=hp) + c2[...], 0.0)
        y = jnp.dot(t, w3[...], preferred_element_type=jnp.float32,
                    precision=hp) + c3[...]
        bvec = b_ref[0]
        gids = lax.broadcasted_iota(jnp.int32, (num_graphs, blk), 0)
        onehot = (gids == bvec).astype(jnp.float32)
        sums = jnp.dot(onehot, y, preferred_element_type=jnp.float32,
                       precision=hp)
        cnts = jnp.sum(onehot, axis=1, keepdims=True)
        upd = jnp.concatenate(
            [sums, cnts, jnp.zeros((num_graphs, 126), jnp.float32)], axis=1)

        @pl.when(i == 0)
        def _():
            out_ref[...] = jnp.zeros_like(out_ref)

        out_ref[...] += upd

        @pl.when(i == grid - 1)
        def _():
            acc = out_ref[...]
            mean = acc[:, 0:1] / jnp.maximum(acc[:, 1:2], 1.0)
            res = 1.0 / (1.0 + jnp.exp(-mean))
            out_ref[...] = jnp.concatenate([acc[:, 0:2], res, acc[:, 3:]], axis=1)

    full = lambda r, c: pl.BlockSpec((r, c), lambda i: (0, 0))
    out = pl.pallas_call(
        body,
        grid=(grid,),
        in_specs=[
            pl.BlockSpec((blk, 32), lambda i: (i, 0)),
            pl.BlockSpec((blk, 32), lambda i: (i, 0)),
            pl.BlockSpec((1, 1, blk), lambda i: (i, 0, 0)),
            full(32, 16), full(1, 16), full(16, 8), full(1, 8), full(8, 1), full(1, 1),
        ],
        out_specs=pl.BlockSpec((num_graphs, 128), lambda i: (0, 0)),
        out_shape=jax.ShapeDtypeStruct((num_graphs, 128), jnp.float32),
    )(agg2, hr, b3d, W1, bb1.reshape(1, 16), W2, bb2.reshape(1, 8), W3,
      bb3.reshape(1, 1))
    return out[:, 2]


def kernel(feat, edge_index, b, W_rel1, b_rel1, W_root1, W_rel2, b_rel2,
           W_root2, W1, bb1, W2, bb2, W3, bb3):
    n = feat.shape[0]
    num_graphs = 64
    blk = 2000
    src = edge_index[0]
    dst = edge_index[1]

    feat16 = jnp.concatenate([feat, jnp.zeros((n, 15), jnp.float32)], axis=1)
    agg1 = _seg_sum_sc(feat16, src, dst, n, 16)[:, :1]
    g, hr = _dense1(feat, agg1, W_rel1, b_rel1, W_root1,
                    W_rel2, b_rel2, W_root2, blk)
    agg2 = _seg_sum_sc(g, src, dst, n, 32)
    b3d = b.reshape(n // blk, 1, blk)
    return _dense2(agg2, hr, b3d, W1, bb1, W2, bb2, W3, bb3, num_graphs, blk)


# speedup vs baseline: 6.6129x; 1.0696x over previous
import jax
import jax.numpy as jnp
from jax import lax
from jax.experimental import pallas as pl
from jax.experimental.pallas import tpu as pltpu
from jax.experimental.pallas import tpu_sc as plsc

NC = 2
NS = 16
NW = NC * NS
CH = 128
DUM = 2048


def _seg_sum_sc(table, src, dst, n, d):
    e = src.shape[0]
    h = n // NC
    epw = e // NS
    nfull = epw // CH
    tail = epw - nfull * CH
    zr = -(-(h + DUM) // (NS * 8)) * 8
    hp = zr * NS
    big = -(-h // NS) // 8 * 8
    while big * (NS - 1) > h:
        big -= 8
    rem = h - big * (NS - 1)
    assert tail % 16 == 0 and rem > 0 and (d * big) % 8 == 0 and (d * rem) % 8 == 0

    zeros = jnp.zeros((zr, d), jnp.float32)

    mesh = plsc.VectorSubcoreMesh(core_axis_name="c", subcore_axis_name="s",
                                  num_cores=NC, num_subcores=NS)

    def body(z_hbm, table_hbm, src_hbm, dst_hbm, out_hbm,
             idx_v, dst_v, rows_v, idx_t, dst_t, rows_t, acc, sem):
        c = lax.axis_index("c")
        s = lax.axis_index("s")
        base_node = c * h

        pltpu.sync_copy(z_hbm, acc.at[pl.ds(s * zr, zr)])
        plsc.subcore_barrier()

        ebase = s * epw

        def remap(dstr, nb):
            for k in range(nb // 16):
                dd = dstr[pl.ds(k * 16, 16)]
                inr = (dd >= base_node) & (dd < base_node + h)
                dl = jnp.where(inr, dd - base_node, h + (dd & (DUM - 1)))
                dstr[pl.ds(k * 16, 16)] = dl

        def chunk(off, nb, idxr, dstr, rowsr):
            off = pl.multiple_of(off, 8)
            pltpu.sync_copy(src_hbm.at[pl.ds(off, nb)], idxr)
            pltpu.sync_copy(dst_hbm.at[pl.ds(off, nb)], dstr)
            remap(dstr, nb)
            pltpu.async_copy(table_hbm.at[idxr], rowsr, sem).wait()
            pltpu.sync_copy(rowsr, acc.at[dstr], add=True)

        def loop_body(i, carry):
            chunk(ebase + i * CH, CH, idx_v, dst_v, rows_v)
            return carry

        lax.fori_loop(0, nfull, loop_body, 0)
        if tail:
            chunk(ebase + nfull * CH, tail, idx_t, dst_t, rows_t)

        plsc.subcore_barrier()
        plsc.subcore_barrier()
        plsc.subcore_barrier()

        @pl.when(s < NS - 1)
        def _():
            o = pl.multiple_of(s * big, 8)
            pltpu.sync_copy(acc.at[pl.ds(o, big)],
                            out_hbm.at[pl.ds(base_node + o, big)])

        @pl.when(s == NS - 1)
        def _():
            o = (NS - 1) * big
            pltpu.sync_copy(acc.at[pl.ds(o, rem)],
                            out_hbm.at[pl.ds(base_node + o, rem)])

    kern = pl.kernel(
        body,
        out_type=jax.ShapeDtypeStruct((n, d), jnp.float32),
        mesh=mesh,
        compiler_params=pltpu.CompilerParams(use_tc_tiling_on_sc=False,
                                             has_side_effects=True),
        scratch_types=[
            pltpu.VMEM((CH,), jnp.int32),
            pltpu.VMEM((CH,), jnp.int32),
            pltpu.VMEM((CH, d), jnp.float32),
            pltpu.VMEM((max(tail, 16),), jnp.int32),
            pltpu.VMEM((max(tail, 16),), jnp.int32),
            pltpu.VMEM((max(tail, 16), d), jnp.float32),
            pltpu.VMEM_SHARED((hp, d), jnp.float32),
            pltpu.SemaphoreType.DMA,
        ],
    )
    return kern(zeros, table, src, dst)


def _dense1(feat, agg1, W_rel1, b_rel1, W_root1, W_rel2, b_rel2, W_root2, blk):
    n = feat.shape[0]
    grid = n // blk

    def body(feat_ref, agg_ref, wr1, br1, wo1, wr2, br2, wo2, g_ref, hr_ref):
        x = feat_ref[...]
        a = agg_ref[...]
        hcur = jnp.maximum(a * wr1[...] + x * wo1[...] + br1[...], 0.0)
        g_ref[...] = jnp.dot(hcur, wr2[...], preferred_element_type=jnp.float32)
        hr_ref[...] = jnp.dot(hcur, wo2[...], preferred_element_type=jnp.float32) + br2[...]

    full = lambda shape: pl.BlockSpec(shape, lambda i: (0, 0))
    return pl.pallas_call(
        body,
        grid=(grid,),
        in_specs=[
            pl.BlockSpec((blk, 1), lambda i: (i, 0)),
            pl.BlockSpec((blk, 1), lambda i: (i, 0)),
            full((1, 64)), full((1, 64)), full((1, 64)),
            full((64, 32)), full((1, 32)), full((64, 32)),
        ],
        out_specs=[pl.BlockSpec((blk, 32), lambda i: (i, 0)),
                   pl.BlockSpec((blk, 32), lambda i: (i, 0))],
        out_shape=[jax.ShapeDtypeStruct((n, 32), jnp.float32),
                   jax.ShapeDtypeStruct((n, 32), jnp.float32)],
    )(feat, agg1, W_rel1.reshape(1, 64), b_rel1.reshape(1, 64), W_root1.reshape(1, 64),
      W_rel2, b_rel2.reshape(1, 32), W_root2)


def _dense2(agg2, hr, b3d, W1, bb1, W2, bb2, W3, bb3, num_graphs, blk):
    n = agg2.shape[0]
    grid = n // blk

    def body(agg_ref, hr_ref, b_ref, w1, c1, w2, c2, w3, c3, out_ref):
        i = pl.program_id(0)
        x2 = jnp.maximum(agg_ref[...] + hr_ref[...], 0.0)
        t = jnp.maximum(jnp.dot(x2, w1[...], preferred_element_type=jnp.float32) + c1[...], 0.0)
        t = jnp.maximum(jnp.dot(t, w2[...], preferred_element_type=jnp.float32) + c2[...], 0.0)
        y = jnp.dot(t, w3[...], preferred_element_type=jnp.float32) + c3[...]
        bvec = b_ref[0]
        gids = lax.broadcasted_iota(jnp.int32, (num_graphs, blk), 0)
        onehot = (gids == bvec).astype(jnp.float32)
        sums = jnp.dot(onehot, y, preferred_element_type=jnp.float32)
        cnts = jnp.sum(onehot, axis=1, keepdims=True)
        upd = jnp.concatenate(
            [sums, cnts, jnp.zeros((num_graphs, 126), jnp.float32)], axis=1)

        @pl.when(i == 0)
        def _():
            out_ref[...] = jnp.zeros_like(out_ref)

        out_ref[...] += upd

        @pl.when(i == grid - 1)
        def _():
            acc = out_ref[...]
            mean = acc[:, 0:1] / jnp.maximum(acc[:, 1:2], 1.0)
            res = 1.0 / (1.0 + jnp.exp(-mean))
            out_ref[...] = jnp.concatenate([acc[:, 0:2], res, acc[:, 3:]], axis=1)

    full = lambda r, c: pl.BlockSpec((r, c), lambda i: (0, 0))
    out = pl.pallas_call(
        body,
        grid=(grid,),
        in_specs=[
            pl.BlockSpec((blk, 32), lambda i: (i, 0)),
            pl.BlockSpec((blk, 32), lambda i: (i, 0)),
            pl.BlockSpec((1, 1, blk), lambda i: (i, 0, 0)),
            full(32, 16), full(1, 16), full(16, 8), full(1, 8), full(8, 1), full(1, 1),
        ],
        out_specs=pl.BlockSpec((num_graphs, 128), lambda i: (0, 0)),
        out_shape=jax.ShapeDtypeStruct((num_graphs, 128), jnp.float32),
    )(agg2, hr, b3d, W1, bb1.reshape(1, 16), W2, bb2.reshape(1, 8), W3,
      bb3.reshape(1, 1))
    return out[:, 2]


def kernel(feat, edge_index, b, W_rel1, b_rel1, W_root1, W_rel2, b_rel2,
           W_root2, W1, bb1, W2, bb2, W3, bb3):
    n = feat.shape[0]
    num_graphs = 64
    blk = 2000
    src = edge_index[0]
    dst = edge_index[1]

    feat16 = jnp.concatenate([feat, jnp.zeros((n, 15), jnp.float32)], axis=1)
    agg1 = _seg_sum_sc(feat16, src, dst, n, 16)[:, :1]
    g, hr = _dense1(feat, agg1, W_rel1, b_rel1, W_root1,
                    W_rel2, b_rel2, W_root2, blk)
    agg2 = _seg_sum_sc(g, src, dst, n, 32)
    b3d = b.reshape(n // blk, 1, blk)
    return _dense2(agg2, hr, b3d, W1, bb1, W2, bb2, W3, bb3, num_graphs, blk)
